# baseline (device time: 12972 ns/iter reference)
import jax
import jax.numpy as jnp
from jax import lax
from jax.experimental import pallas as pl
from jax.experimental.pallas import tpu as pltpu

T = 256
V_SHARD = 4096


def kernel(x, W, labels):

    def body(x_ref, w_ref, labels_ref, out_ref, stats_ref, rstats_ref,
             send_sem, recv_sem):
        my_x = lax.axis_index("x")
        my_y = lax.axis_index("y")
        my_z = lax.axis_index("z")
        partner = (1 - my_x, my_y, my_z)

        barrier_sem = pltpu.get_barrier_semaphore()
        pl.semaphore_signal(
            barrier_sem, inc=1,
            device_id=partner, device_id_type=pl.DeviceIdType.MESH,
        )
        pl.semaphore_wait(barrier_sem, 1)

        logits = jnp.dot(
            x_ref[:, :], w_ref[:, :], preferred_element_type=jnp.float32
        )

        m = jnp.max(logits, axis=1)
        s = jnp.sum(jnp.exp(logits - m[:, None]), axis=1)
        local_idx = labels_ref[:] - my_x * V_SHARD
        col = lax.broadcasted_iota(jnp.int32, (T, V_SHARD), 1)
        ll = jnp.sum(
            jnp.where(col == local_idx[:, None], logits, 0.0), axis=1
        )

        stats_ref[0, :] = m
        stats_ref[1, :] = s
        stats_ref[2, :] = ll

        rdma = pltpu.make_async_remote_copy(
            src_ref=stats_ref,
            dst_ref=rstats_ref,
            send_sem=send_sem,
            recv_sem=recv_sem,
            device_id=partner,
            device_id_type=pl.DeviceIdType.MESH,
        )
        rdma.start()
        rdma.wait()

        m2 = rstats_ref[0, :]
        s2 = rstats_ref[1, :]
        ll2 = rstats_ref[2, :]
        gm = jnp.maximum(m, m2)
        gs = s * jnp.exp(m - gm) + s2 * jnp.exp(m2 - gm)
        out_ref[:] = gm + jnp.log(gs) - (ll + ll2)

    return pl.pallas_call(
        body,
        out_shape=jax.ShapeDtypeStruct((T,), jnp.float32),
        in_specs=[
            pl.BlockSpec(memory_space=pltpu.VMEM),
            pl.BlockSpec(memory_space=pltpu.VMEM),
            pl.BlockSpec(memory_space=pltpu.VMEM),
        ],
        out_specs=pl.BlockSpec(memory_space=pltpu.VMEM),
        scratch_shapes=[
            pltpu.VMEM((3, T), jnp.float32),
            pltpu.VMEM((3, T), jnp.float32),
            pltpu.SemaphoreType.DMA,
            pltpu.SemaphoreType.DMA,
        ],
        compiler_params=pltpu.CompilerParams(collective_id=0),
    )(x, W, labels)


# device time: 11756 ns/iter; 1.1034x vs baseline; 1.1034x over previous
import jax
import jax.numpy as jnp
from jax import lax
from jax.experimental import pallas as pl
from jax.experimental.pallas import tpu as pltpu

T = 256
V_SHARD = 4096
N_CHUNKS = 4
VC = V_SHARD // N_CHUNKS


def kernel(x, W, labels):

    def body(x_ref, w_ref, labels_ref, out_ref, stats_ref, rstats_ref,
             send_sem, recv_sem):
        my_x = lax.axis_index("x")
        my_y = lax.axis_index("y")
        my_z = lax.axis_index("z")
        partner = (1 - my_x, my_y, my_z)

        barrier_sem = pltpu.get_barrier_semaphore()
        pl.semaphore_signal(
            barrier_sem, inc=1,
            device_id=partner, device_id_type=pl.DeviceIdType.MESH,
        )

        xv = x_ref[:, :]
        local_idx = labels_ref[:] - my_x * V_SHARD

        s = jnp.zeros((T,), jnp.float32)
        ll = jnp.zeros((T,), jnp.float32)
        for c in range(N_CHUNKS):
            logits = jnp.dot(
                xv, w_ref[:, c * VC:(c + 1) * VC],
                preferred_element_type=jnp.float32,
            )
            s = s + jnp.sum(jnp.exp(logits), axis=1)
            col = c * VC + lax.broadcasted_iota(jnp.int32, (T, VC), 1)
            ll = ll + jnp.sum(
                jnp.where(col == local_idx[:, None], logits, 0.0), axis=1
            )

        stats_ref[0, :] = s
        stats_ref[1, :] = ll

        pl.semaphore_wait(barrier_sem, 1)
        rdma = pltpu.make_async_remote_copy(
            src_ref=stats_ref,
            dst_ref=rstats_ref,
            send_sem=send_sem,
            recv_sem=recv_sem,
            device_id=partner,
            device_id_type=pl.DeviceIdType.MESH,
        )
        rdma.start()
        rdma.wait()

        out_ref[:] = (
            jnp.log(stats_ref[0, :] + rstats_ref[0, :])
            - (stats_ref[1, :] + rstats_ref[1, :])
        )

    return pl.pallas_call(
        body,
        out_shape=jax.ShapeDtypeStruct((T,), jnp.float32),
        in_specs=[
            pl.BlockSpec(memory_space=pltpu.VMEM),
            pl.BlockSpec(memory_space=pltpu.VMEM),
            pl.BlockSpec(memory_space=pltpu.VMEM),
        ],
        out_specs=pl.BlockSpec(memory_space=pltpu.VMEM),
        scratch_shapes=[
            pltpu.VMEM((2, T), jnp.float32),
            pltpu.VMEM((2, T), jnp.float32),
            pltpu.SemaphoreType.DMA,
            pltpu.SemaphoreType.DMA,
        ],
        compiler_params=pltpu.CompilerParams(collective_id=0),
    )(x, W, labels)


# device time: 9443 ns/iter; 1.3737x vs baseline; 1.2449x over previous
import jax
import jax.numpy as jnp
from jax import lax
from jax.experimental import pallas as pl
from jax.experimental.pallas import tpu as pltpu

T = 256
D = 512
V_SHARD = 4096
N_CHUNKS = 4
VC = V_SHARD // N_CHUNKS


def kernel(x, W, labels):

    def body(x_hbm, w_hbm, labels_hbm, out_ref,
             x_vmem, w_vmem, labels_vmem, stats_ref, rstats_ref,
             in_sems, send_sem, recv_sem):
        my_x = lax.axis_index("x")
        my_y = lax.axis_index("y")
        my_z = lax.axis_index("z")
        partner = (1 - my_x, my_y, my_z)

        barrier_sem = pltpu.get_barrier_semaphore()
        pl.semaphore_signal(
            barrier_sem, inc=1,
            device_id=partner, device_id_type=pl.DeviceIdType.MESH,
        )

        cp_x = pltpu.make_async_copy(x_hbm, x_vmem, in_sems.at[N_CHUNKS])
        cp_lab = pltpu.make_async_copy(
            labels_hbm, labels_vmem, in_sems.at[N_CHUNKS + 1]
        )
        cp_x.start()
        cp_lab.start()
        cp_w = []
        for c in range(N_CHUNKS):
            sl = pl.ds(c * VC, VC)
            cp = pltpu.make_async_copy(
                w_hbm.at[:, sl], w_vmem.at[:, sl], in_sems.at[c]
            )
            cp.start()
            cp_w.append(cp)

        cp_x.wait()
        cp_lab.wait()
        xv = x_vmem[:, :]
        local_idx = labels_vmem[:] - my_x * V_SHARD

        s = jnp.zeros((T,), jnp.float32)
        ll = jnp.zeros((T,), jnp.float32)
        for c in range(N_CHUNKS):
            cp_w[c].wait()
            logits = jnp.dot(
                xv, w_vmem[:, c * VC:(c + 1) * VC],
                preferred_element_type=jnp.float32,
            )
            s = s + jnp.sum(jnp.exp(logits), axis=1)
            col = c * VC + lax.broadcasted_iota(jnp.int32, (T, VC), 1)
            ll = ll + jnp.sum(
                jnp.where(col == local_idx[:, None], logits, 0.0), axis=1
            )

        stats_ref[0, :] = s
        stats_ref[1, :] = ll

        pl.semaphore_wait(barrier_sem, 1)
        rdma = pltpu.make_async_remote_copy(
            src_ref=stats_ref,
            dst_ref=rstats_ref,
            send_sem=send_sem,
            recv_sem=recv_sem,
            device_id=partner,
            device_id_type=pl.DeviceIdType.MESH,
        )
        rdma.start()
        rdma.wait()

        out_ref[:] = (
            jnp.log(stats_ref[0, :] + rstats_ref[0, :])
            - (stats_ref[1, :] + rstats_ref[1, :])
        )

    x = pltpu.with_memory_space_constraint(x, pltpu.HBM)
    W = pltpu.with_memory_space_constraint(W, pltpu.HBM)
    labels = pltpu.with_memory_space_constraint(labels, pltpu.HBM)

    return pl.pallas_call(
        body,
        out_shape=jax.ShapeDtypeStruct((T,), jnp.float32),
        in_specs=[
            pl.BlockSpec(memory_space=pltpu.HBM),
            pl.BlockSpec(memory_space=pltpu.HBM),
            pl.BlockSpec(memory_space=pltpu.HBM),
        ],
        out_specs=pl.BlockSpec(memory_space=pltpu.VMEM),
        scratch_shapes=[
            pltpu.VMEM((T, D), jnp.float32),
            pltpu.VMEM((D, V_SHARD), jnp.float32),
            pltpu.VMEM((T,), jnp.int32),
            pltpu.VMEM((2, T), jnp.float32),
            pltpu.VMEM((2, T), jnp.float32),
            pltpu.SemaphoreType.DMA((N_CHUNKS + 2,)),
            pltpu.SemaphoreType.DMA,
            pltpu.SemaphoreType.DMA,
        ],
        compiler_params=pltpu.CompilerParams(collective_id=0),
    )(x, W, labels)
